# Initial kernel scaffold; baseline (speedup 1.0000x reference)
import jax
import jax.numpy as jnp
from jax.experimental import pallas as pl


def kernel(row_idx, col_idx, values, kernel, bias, dense_w, dense_b):
    raise NotImplementedError("write your pallas kernel here")


# speedup vs baseline: 7.3155x; 7.3155x over previous
import functools

import jax
import jax.numpy as jnp
from jax import lax
from jax.experimental import pallas as pl
from jax.experimental.pallas import tpu as pltpu
from jax.experimental.pallas import tpu_sc as plsc

NC = 2
NS = 16
NW = NC * NS
L = 16

B = 16384
CH = 256
RG = 32


def _sc_partials(row2, col_idx, values, table_flat, zeros, n_dims):
    nnz = col_idx.shape[0]
    H = n_dims
    VH = table_flat.shape[0]
    per_w = nnz // NW
    n_chunks = per_w // CH
    assert per_w % CH == 0 and CH % RG == 0 and B % NS == 0

    mesh = plsc.VectorSubcoreMesh(
        core_axis_name="c", subcore_axis_name="s", num_cores=NC, num_subcores=NS
    )

    @functools.partial(
        pl.kernel,
        out_type=jax.ShapeDtypeStruct((NC, B, H), jnp.float32),
        mesh=mesh,
        scratch_types=[
            pltpu.VMEM((VH,), jnp.float32),
            pltpu.VMEM((CH,), jnp.int32),
            pltpu.VMEM((CH,), jnp.float32),
            pltpu.VMEM((CH // RG, RG), jnp.int32),
            pltpu.VMEM((CH, H), jnp.float32),
            pltpu.VMEM_SHARED((B, H), jnp.float32),
            pltpu.SemaphoreType.DMA,
            pltpu.SemaphoreType.DMA,
            pltpu.SemaphoreType.DMA,
            pltpu.SemaphoreType.DMA,
        ],
        compiler_params=pltpu.CompilerParams(use_tc_tiling_on_sc=False),
    )
    def sc_kernel(row_hbm, col_hbm, val_hbm, table_hbm, zeros_hbm, out_hbm,
                  table_v, col_v, val_v, row_v, buf, acc_sh,
                  sem0, sem1, sem2, sem3):
        c = lax.axis_index("c")
        s = lax.axis_index("s")
        w = c * NS + s

        tcopy = pltpu.async_copy(table_hbm, table_v, sem3)
        rows_per_tile = B // NS
        pltpu.sync_copy(
            zeros_hbm.at[pl.ds(s * rows_per_tile, rows_per_tile), :],
            acc_sh.at[pl.ds(s * rows_per_tile, rows_per_tile), :],
        )
        tcopy.wait()
        plsc.subcore_barrier()

        def chunk_body(k, _):
            base = w * per_w + k * CH
            cp0 = pltpu.async_copy(col_hbm.at[pl.ds(base, CH)], col_v, sem0)
            cp1 = pltpu.async_copy(val_hbm.at[pl.ds(base, CH)], val_v, sem1)
            cp2 = pltpu.async_copy(
                row_hbm.at[pl.ds(pl.multiple_of(base // RG, 8), CH // RG), :],
                row_v,
                sem2,
            )
            cp0.wait()
            cp1.wait()

            def g_body(g, _):
                cvecw = col_v[pl.ds(g * L, L)] * H
                vvec = val_v[pl.ds(g * L, L)]
                for jj in range(L):
                    colw = cvecw[jj]
                    val = vvec[jj]
                    i = g * L + jj
                    for j in range(H // L):
                        buf[i, pl.ds(j * L, L)] = (
                            table_v[pl.ds(colw + j * L, L)] * val
                        )
                return 0

            lax.fori_loop(0, CH // L, g_body, 0)
            cp2.wait()
            for gg in range(CH // RG):
                pltpu.sync_copy(
                    buf.at[pl.ds(gg * RG, RG), :],
                    acc_sh.at[row_v.at[gg]],
                    add=True,
                )
            return 0

        lax.fori_loop(0, n_chunks, chunk_body, 0)

        plsc.subcore_barrier()
        pltpu.sync_copy(
            acc_sh.at[pl.ds(s * rows_per_tile, rows_per_tile), :],
            out_hbm.at[c, pl.ds(s * rows_per_tile, rows_per_tile), :],
        )

    return sc_kernel(row2, col_idx, values, table_flat, zeros)


def _head(partials, bias, dense_w, dense_b):
    _, b_rows, H = partials.shape
    n_cls = dense_w.shape[1]
    BLK = 2048

    def head_kernel(p_ref, bias_ref, w_ref, b_ref, o_ref):
        h = p_ref[0] + p_ref[1] + bias_ref[...]
        mask = jnp.where(h > 0.0, 1.0, 0.5)
        h = jnp.maximum(h * mask, 0.0)
        o_ref[...] = (
            jnp.dot(h, w_ref[...], preferred_element_type=jnp.float32)
            + b_ref[...]
        )

    return pl.pallas_call(
        head_kernel,
        grid=(b_rows // BLK,),
        in_specs=[
            pl.BlockSpec((NC, BLK, H), lambda i: (0, i, 0)),
            pl.BlockSpec((1, H), lambda i: (0, 0)),
            pl.BlockSpec((H, n_cls), lambda i: (0, 0)),
            pl.BlockSpec((1, n_cls), lambda i: (0, 0)),
        ],
        out_specs=pl.BlockSpec((BLK, n_cls), lambda i: (i, 0)),
        out_shape=jax.ShapeDtypeStruct((b_rows, n_cls), jnp.float32),
    )(partials, bias, dense_w, dense_b)


def kernel(row_idx, col_idx, values, kernel, bias, dense_w, dense_b):
    nnz = row_idx.shape[0]
    V, H = kernel.shape
    row2 = row_idx.reshape(nnz // RG, RG)
    zeros = jnp.zeros((B, H), jnp.float32)
    partials = _sc_partials(row2, col_idx, values, kernel.reshape(V * H),
                            zeros, H)
    return _head(
        partials,
        bias.reshape(1, -1),
        dense_w,
        dense_b.reshape(1, -1),
    )
